# initial kernel scaffold (unmeasured)
import jax
import jax.numpy as jnp
from jax import lax
from jax.experimental import pallas as pl
from jax.experimental.pallas import tpu as pltpu


def kernel(
    x,
):
    def body(*refs):
        pass

    out_shape = jax.ShapeDtypeStruct(..., jnp.float32)
    return pl.pallas_call(body, out_shape=out_shape)(...)


# baseline (device time: 618010 ns/iter reference)
import jax
import jax.numpy as jnp
from jax import lax
from jax.experimental import pallas as pl
from jax.experimental.pallas import tpu as pltpu

NZ = 4


def kernel(x):
    x = x.astype(jnp.bfloat16)
    m_per, n = x.shape

    def body(x_ref, out_ref, copy_sem, send_sems, recv_sems):
        my_x = lax.axis_index("x")
        my_y = lax.axis_index("y")
        my_z = lax.axis_index("z")
        right = (my_z + 1) % NZ

        local = pltpu.make_async_copy(
            x_ref, out_ref.at[pl.ds(my_z * m_per, m_per), :], copy_sem
        )
        local.start()
        local.wait()

        for h in range(NZ - 1):
            origin = (my_z - h) % NZ
            src = out_ref.at[pl.ds(origin * m_per, m_per), :]
            rdma = pltpu.make_async_remote_copy(
                src_ref=src,
                dst_ref=src,
                send_sem=send_sems.at[h],
                recv_sem=recv_sems.at[h],
                device_id=(my_x, my_y, right),
                device_id_type=pl.DeviceIdType.MESH,
            )
            rdma.start()
            rdma.wait()

    return pl.pallas_call(
        body,
        out_shape=jax.ShapeDtypeStruct((NZ * m_per, n), jnp.bfloat16),
        in_specs=[pl.BlockSpec(memory_space=pl.ANY)],
        out_specs=pl.BlockSpec(memory_space=pl.ANY),
        scratch_shapes=[
            pltpu.SemaphoreType.DMA,
            pltpu.SemaphoreType.DMA((NZ - 1,)),
            pltpu.SemaphoreType.DMA((NZ - 1,)),
        ],
    )(x)


# device time: 363378 ns/iter; 1.7007x vs baseline; 1.7007x over previous
import jax
import jax.numpy as jnp
from jax import lax
from jax.experimental import pallas as pl
from jax.experimental.pallas import tpu as pltpu

NZ = 4
S = 4


def kernel(x):
    x = x.astype(jnp.bfloat16)
    m_per, n = x.shape
    half = m_per // 2
    seg = half // S

    def body(x_ref, out_ref, copy_sem,
             szr, szl, rzr, rzl, sxr, sxl, rxr, rxl):
        my_x = lax.axis_index("x")
        my_y = lax.axis_index("y")
        my_z = lax.axis_index("z")

        right = (my_x, my_y, my_z + 1)
        left = (my_x, my_y, my_z - 1)
        partner = (1 - my_x, my_y, my_z)

        has_right = my_z < NZ - 1
        has_left = my_z > 0

        def sl(c, s_, parity):
            return out_ref.at[pl.ds(c * m_per + parity * half + s_ * seg, seg), :]

        def send(src, dst, ssem, rsem, dev):
            pltpu.make_async_remote_copy(
                src_ref=src, dst_ref=dst, send_sem=ssem, recv_sem=rsem,
                device_id=dev, device_id_type=pl.DeviceIdType.MESH,
            ).start()

        def wait_recv(dst, rsem):
            pltpu.make_async_remote_copy(
                src_ref=dst, dst_ref=dst, send_sem=copy_sem, recv_sem=rsem,
                device_id=partner, device_id_type=pl.DeviceIdType.MESH,
            ).wait_recv()

        def wait_send(src, ssem):
            pltpu.make_async_remote_copy(
                src_ref=src, dst_ref=src, send_sem=ssem, recv_sem=copy_sem,
                device_id=partner, device_id_type=pl.DeviceIdType.MESH,
            ).wait_send()

        cp = pltpu.make_async_copy(
            x_ref, out_ref.at[pl.ds(my_z * m_per, m_per), :], copy_sem
        )
        cp.start()

        for s_ in range(S):
            src = x_ref.at[pl.ds(my_x * half + s_ * seg, seg), :]

            @pl.when(has_right)
            def _():
                send(src, sl(my_z, s_, my_x), szr.at[0, s_], rzr.at[0, s_], right)

            @pl.when(has_left)
            def _():
                send(src, sl(my_z, s_, my_x), szl.at[0, s_], rzl.at[0, s_], left)

        for idx in range(NZ - 1):
            for s_ in range(S):
                r_ev = idx < my_z
                l_ev = idx < NZ - 1 - my_z

                @pl.when(r_ev)
                def _():
                    c = my_z - 1 - idx
                    dst = sl(c, s_, my_x)
                    wait_recv(dst, rzr.at[idx, s_])

                    @pl.when(has_right)
                    def _():
                        send(dst, dst, szr.at[idx + 1, s_],
                             rzr.at[idx + 1, s_], right)

                    send(dst, dst, sxr.at[idx, s_], rxr.at[idx, s_], partner)

                @pl.when(l_ev)
                def _():
                    c = my_z + 1 + idx
                    dst = sl(c, s_, my_x)
                    wait_recv(dst, rzl.at[idx, s_])

                    @pl.when(has_left)
                    def _():
                        send(dst, dst, szl.at[idx + 1, s_],
                             rzl.at[idx + 1, s_], left)

                    send(dst, dst, sxl.at[idx, s_], rxl.at[idx, s_], partner)

        cp.wait()

        for idx in range(NZ - 1):
            for s_ in range(S):
                @pl.when(idx < my_z)
                def _():
                    wait_recv(sl(my_z - 1 - idx, s_, 1 - my_x), rxr.at[idx, s_])

                @pl.when(idx < NZ - 1 - my_z)
                def _():
                    wait_recv(sl(my_z + 1 + idx, s_, 1 - my_x), rxl.at[idx, s_])

        for idx in range(NZ - 1):
            for s_ in range(S):
                r_ev = idx < my_z
                l_ev = idx < NZ - 1 - my_z
                own_src = x_ref.at[pl.ds(my_x * half + s_ * seg, seg), :]

                @pl.when(has_right if idx == 0 else jnp.logical_and(has_right, idx - 1 < my_z))
                def _():
                    wait_send(own_src, szr.at[idx, s_])

                @pl.when(has_left if idx == 0 else jnp.logical_and(has_left, idx - 1 < NZ - 1 - my_z))
                def _():
                    wait_send(own_src, szl.at[idx, s_])

                @pl.when(r_ev)
                def _():
                    wait_send(own_src, sxr.at[idx, s_])

                @pl.when(l_ev)
                def _():
                    wait_send(own_src, sxl.at[idx, s_])

    sems3 = pltpu.SemaphoreType.DMA((NZ - 1, S))
    return pl.pallas_call(
        body,
        out_shape=jax.ShapeDtypeStruct((NZ * m_per, n), jnp.bfloat16),
        in_specs=[pl.BlockSpec(memory_space=pl.ANY)],
        out_specs=pl.BlockSpec(memory_space=pl.ANY),
        scratch_shapes=[
            pltpu.SemaphoreType.DMA,
            sems3, sems3,
            sems3, sems3,
            sems3, sems3,
            sems3, sems3,
        ],
    )(x)


# device time: 345951 ns/iter; 1.7864x vs baseline; 1.0504x over previous
import jax
import jax.numpy as jnp
from jax import lax
from jax.experimental import pallas as pl
from jax.experimental.pallas import tpu as pltpu

NZ = 4
S = 8


def kernel(x):
    x = x.astype(jnp.bfloat16)
    m_per, n = x.shape
    half = m_per // 2
    seg = half // S

    def body(x_ref, out_ref, copy_sem,
             szr, szl, rzr, rzl, sxr, sxl, rxr, rxl):
        my_x = lax.axis_index("x")
        my_y = lax.axis_index("y")
        my_z = lax.axis_index("z")

        right = (my_x, my_y, my_z + 1)
        left = (my_x, my_y, my_z - 1)
        partner = (1 - my_x, my_y, my_z)

        has_right = my_z < NZ - 1
        has_left = my_z > 0

        def sl(c, s_, parity):
            return out_ref.at[pl.ds(c * m_per + parity * half + s_ * seg, seg), :]

        def send(src, dst, ssem, rsem, dev):
            pltpu.make_async_remote_copy(
                src_ref=src, dst_ref=dst, send_sem=ssem, recv_sem=rsem,
                device_id=dev, device_id_type=pl.DeviceIdType.MESH,
            ).start()

        def wait_recv(dst, rsem):
            pltpu.make_async_remote_copy(
                src_ref=dst, dst_ref=dst, send_sem=copy_sem, recv_sem=rsem,
                device_id=partner, device_id_type=pl.DeviceIdType.MESH,
            ).wait_recv()

        def wait_send(src, ssem):
            pltpu.make_async_remote_copy(
                src_ref=src, dst_ref=src, send_sem=ssem, recv_sem=copy_sem,
                device_id=partner, device_id_type=pl.DeviceIdType.MESH,
            ).wait_send()

        barrier_sem = pltpu.get_barrier_semaphore()
        pl.semaphore_signal(barrier_sem, inc=1, device_id=partner,
                            device_id_type=pl.DeviceIdType.MESH)

        @pl.when(has_right)
        def _():
            pl.semaphore_signal(barrier_sem, inc=1, device_id=right,
                                device_id_type=pl.DeviceIdType.MESH)

        @pl.when(has_left)
        def _():
            pl.semaphore_signal(barrier_sem, inc=1, device_id=left,
                                device_id_type=pl.DeviceIdType.MESH)

        n_nbrs = 1 + has_right.astype(jnp.int32) + has_left.astype(jnp.int32)
        pl.semaphore_wait(barrier_sem, n_nbrs)

        cp = pltpu.make_async_copy(
            x_ref, out_ref.at[pl.ds(my_z * m_per, m_per), :], copy_sem
        )
        cp.start()

        for s_ in range(S):
            src = x_ref.at[pl.ds(my_x * half + s_ * seg, seg), :]

            @pl.when(has_right)
            def _():
                send(src, sl(my_z, s_, my_x), szr.at[0, s_], rzr.at[0, s_], right)

            @pl.when(has_left)
            def _():
                send(src, sl(my_z, s_, my_x), szl.at[0, s_], rzl.at[0, s_], left)

        for idx in range(NZ - 1):
            for s_ in range(S):
                r_ev = idx < my_z
                l_ev = idx < NZ - 1 - my_z

                @pl.when(r_ev)
                def _():
                    c = my_z - 1 - idx
                    dst = sl(c, s_, my_x)
                    wait_recv(dst, rzr.at[idx, s_])

                    @pl.when(has_right)
                    def _():
                        send(dst, dst, szr.at[idx + 1, s_],
                             rzr.at[idx + 1, s_], right)

                    send(dst, dst, sxr.at[idx, s_], rxr.at[idx, s_], partner)

                @pl.when(l_ev)
                def _():
                    c = my_z + 1 + idx
                    dst = sl(c, s_, my_x)
                    wait_recv(dst, rzl.at[idx, s_])

                    @pl.when(has_left)
                    def _():
                        send(dst, dst, szl.at[idx + 1, s_],
                             rzl.at[idx + 1, s_], left)

                    send(dst, dst, sxl.at[idx, s_], rxl.at[idx, s_], partner)

        cp.wait()

        for idx in range(NZ - 1):
            for s_ in range(S):
                @pl.when(idx < my_z)
                def _():
                    wait_recv(sl(my_z - 1 - idx, s_, 1 - my_x), rxr.at[idx, s_])

                @pl.when(idx < NZ - 1 - my_z)
                def _():
                    wait_recv(sl(my_z + 1 + idx, s_, 1 - my_x), rxl.at[idx, s_])

        for idx in range(NZ - 1):
            for s_ in range(S):
                r_ev = idx < my_z
                l_ev = idx < NZ - 1 - my_z
                own_src = x_ref.at[pl.ds(my_x * half + s_ * seg, seg), :]

                @pl.when(has_right if idx == 0 else jnp.logical_and(has_right, idx - 1 < my_z))
                def _():
                    wait_send(own_src, szr.at[idx, s_])

                @pl.when(has_left if idx == 0 else jnp.logical_and(has_left, idx - 1 < NZ - 1 - my_z))
                def _():
                    wait_send(own_src, szl.at[idx, s_])

                @pl.when(r_ev)
                def _():
                    wait_send(own_src, sxr.at[idx, s_])

                @pl.when(l_ev)
                def _():
                    wait_send(own_src, sxl.at[idx, s_])

    sems3 = pltpu.SemaphoreType.DMA((NZ - 1, S))
    return pl.pallas_call(
        body,
        out_shape=jax.ShapeDtypeStruct((NZ * m_per, n), jnp.bfloat16),
        in_specs=[pl.BlockSpec(memory_space=pl.ANY)],
        out_specs=pl.BlockSpec(memory_space=pl.ANY),
        scratch_shapes=[
            pltpu.SemaphoreType.DMA,
            sems3, sems3,
            sems3, sems3,
            sems3, sems3,
            sems3, sems3,
        ],
        compiler_params=pltpu.CompilerParams(collective_id=0),
    )(x)
